# baseline (speedup 1.0000x reference)
import functools

import jax
import jax.numpy as jnp
from jax import lax
from jax.experimental import pallas as pl
from jax.experimental.pallas import tpu as pltpu
from jax.experimental.pallas import tpu_sc as plsc

TM = 64
SC_CHUNK = 64
NUM_WORKERS = 32


def _routing_metadata(ei, ew, E, S, nt_max):
    p = jnp.arange(S, dtype=jnp.int32)
    ei_s, perm, ew_s = lax.sort((ei, p, ew), num_keys=1)

    counts = jnp.bincount(ei, length=E).astype(jnp.int32)
    cumc = jnp.cumsum(counts).astype(jnp.int32)
    offs = cumc - counts
    tiles_e = (counts + TM - 1) // TM
    cumtiles = jnp.cumsum(tiles_e).astype(jnp.int32)
    tstart = cumtiles - tiles_e
    num_active = cumtiles[-1]

    tile_ids = jnp.arange(nt_max, dtype=jnp.int32)
    te = jnp.searchsorted(cumtiles, tile_ids, side="right").astype(jnp.int32)
    e_last = jnp.argmax(jnp.where(counts > 0, jnp.arange(E), -1)).astype(jnp.int32)
    te = jnp.where(tile_ids < num_active, te, e_last)
    active = (tile_ids < num_active).astype(jnp.int32)

    base = tstart * TM - offs
    row_p = base[ei_s] + p

    R = nt_max * TM
    fill = jnp.broadcast_to(p, (R // S, S)).reshape(R)
    src = fill.at[row_p].set(perm)
    w_pad = jnp.zeros((R,), ew.dtype).at[row_p].set(ew_s)
    r_of_t = jnp.zeros((S,), jnp.int32).at[perm].set(row_p)
    return te, active, src, w_pad, r_of_t


def _sc_gather_rows(table, idx):
    R = idx.shape[0]
    H = table.shape[1]
    rows_pw = R // NUM_WORKERS
    nch = rows_pw // SC_CHUNK
    mesh = plsc.VectorSubcoreMesh(core_axis_name="c", subcore_axis_name="s")

    @functools.partial(
        pl.kernel,
        out_type=jax.ShapeDtypeStruct((R, H), jnp.float32),
        mesh=mesh,
        scratch_types=[
            pltpu.VMEM((SC_CHUNK,), jnp.int32),
            pltpu.VMEM((SC_CHUNK, H), jnp.float32),
            pltpu.SemaphoreType.DMA,
        ],
    )
    def gather_kernel(table_hbm, idx_hbm, out_hbm, idx_v, rows_v, sem):
        wid = lax.axis_index("s") * 2 + lax.axis_index("c")
        base = wid * rows_pw
        for c in range(nch):
            off = base + c * SC_CHUNK
            pltpu.sync_copy(idx_hbm.at[pl.ds(off, SC_CHUNK)], idx_v)
            pltpu.async_copy(table_hbm.at[idx_v], rows_v, sem).wait()
            pltpu.sync_copy(rows_v, out_hbm.at[pl.ds(off, SC_CHUNK)])

    return gather_kernel(table, idx)


NFF = 1


def _ffn_body(te_ref, act_ref, x_ref, g_ref, u_ref, d_ref, w_ref, o_ref):
    i = pl.program_id(0)
    f = pl.program_id(1)

    @pl.when(act_ref[i] == 1)
    def _():
        x = x_ref[...]
        g = lax.dot_general(x, g_ref[0], (((1,), (1,)), ((), ())),
                            preferred_element_type=jnp.float32)
        u = lax.dot_general(x, u_ref[0], (((1,), (1,)), ((), ())),
                            preferred_element_type=jnp.float32)
        inter = g * jax.nn.sigmoid(g) * u
        y = lax.dot_general(inter, d_ref[0], (((1,), (1,)), ((), ())),
                            preferred_element_type=jnp.float32)
        y = y * w_ref[0, 0][:, None]

        @pl.when(f == 0)
        def _():
            o_ref[...] = y

        @pl.when(f != 0)
        def _():
            o_ref[...] += y


def _grouped_ffn(x_pad, gate, up, down, w_pad, te, active, nt_max, interpret=False):
    E, FF, H = gate.shape
    R = nt_max * TM
    FB = FF // NFF
    w3 = w_pad.reshape(nt_max, 1, TM)
    grid_spec = pltpu.PrefetchScalarGridSpec(
        num_scalar_prefetch=2,
        grid=(nt_max, NFF),
        in_specs=[
            pl.BlockSpec((TM, H), lambda i, f, te, act: (i, 0)),
            pl.BlockSpec((1, FB, H), lambda i, f, te, act: (te[i], f, 0)),
            pl.BlockSpec((1, FB, H), lambda i, f, te, act: (te[i], f, 0)),
            pl.BlockSpec((1, H, FB), lambda i, f, te, act: (te[i], 0, f)),
            pl.BlockSpec((1, 1, TM), lambda i, f, te, act: (i, 0, 0)),
        ],
        out_specs=pl.BlockSpec((TM, H), lambda i, f, te, act: (i, 0)),
    )
    return pl.pallas_call(
        _ffn_body,
        grid_spec=grid_spec,
        out_shape=jax.ShapeDtypeStruct((R, H), jnp.float32),
        interpret=interpret,
    )(te, active, x_pad, gate, up, down, w3)


def kernel(x, expert_indices, expert_weights, gate_proj, up_proj, down_proj):
    B, S, H = x.shape
    E, FF, _ = gate_proj.shape
    nt_max = S // TM + E

    x_flat = x.reshape(S, H)
    ei = expert_indices.reshape(S).astype(jnp.int32)
    ew = expert_weights.reshape(S)

    te, active, src, w_pad, r_of_t = _routing_metadata(ei, ew, E, S, nt_max)
    x_padded = _sc_gather_rows(x_flat, src)
    y_padded = _grouped_ffn(x_padded, gate_proj, up_proj, down_proj,
                            w_pad, te, active, nt_max)
    out = _sc_gather_rows(y_padded, r_of_t)
    return out.reshape(B, S, H)


# speedup vs baseline: 1.0283x; 1.0283x over previous
import functools

import jax
import jax.numpy as jnp
from jax import lax
from jax.experimental import pallas as pl
from jax.experimental.pallas import tpu as pltpu
from jax.experimental.pallas import tpu_sc as plsc

TM = 64
SC_CHUNK = 64
NUM_WORKERS = 32


def _routing_metadata(ei, ew, E, S, nt_max):
    p = jnp.arange(S, dtype=jnp.int32)
    ei_s, perm, ew_s = lax.sort((ei, p, ew), num_keys=1)

    counts = jnp.bincount(ei, length=E).astype(jnp.int32)
    cumc = jnp.cumsum(counts).astype(jnp.int32)
    offs = cumc - counts
    tiles_e = (counts + TM - 1) // TM
    cumtiles = jnp.cumsum(tiles_e).astype(jnp.int32)
    tstart = cumtiles - tiles_e
    num_active = cumtiles[-1]

    tile_ids = jnp.arange(nt_max, dtype=jnp.int32)
    te = jnp.searchsorted(cumtiles, tile_ids, side="right").astype(jnp.int32)
    e_last = jnp.argmax(jnp.where(counts > 0, jnp.arange(E), -1)).astype(jnp.int32)
    te = jnp.where(tile_ids < num_active, te, e_last)
    active = (tile_ids < num_active).astype(jnp.int32)

    base = tstart * TM - offs
    row_p = base[ei_s] + p

    R = nt_max * TM
    w_pad = jnp.zeros((R,), ew.dtype).at[row_p].set(ew_s)
    return te, active, perm, row_p, w_pad


def _sc_permute_rows(table, src_idx, dst_idx, out_rows):
    S = src_idx.shape[0]
    H = table.shape[1]
    rows_pw = S // NUM_WORKERS
    nch = rows_pw // SC_CHUNK
    mesh = plsc.VectorSubcoreMesh(core_axis_name="c", subcore_axis_name="s")

    @functools.partial(
        pl.kernel,
        out_type=jax.ShapeDtypeStruct((out_rows, H), jnp.float32),
        mesh=mesh,
        scratch_types=[
            pltpu.VMEM((SC_CHUNK,), jnp.int32),
            pltpu.VMEM((SC_CHUNK,), jnp.int32),
            pltpu.VMEM((SC_CHUNK, H), jnp.float32),
            pltpu.SemaphoreType.DMA,
            pltpu.SemaphoreType.DMA,
        ],
    )
    def permute_kernel(table_hbm, si_hbm, di_hbm, out_hbm,
                       si_v, di_v, rows_v, sem_g, sem_s):
        wid = lax.axis_index("s") * 2 + lax.axis_index("c")
        for c in range(nch):
            off = (wid * nch + c) * SC_CHUNK
            pltpu.sync_copy(si_hbm.at[pl.ds(off, SC_CHUNK)], si_v)
            pltpu.sync_copy(di_hbm.at[pl.ds(off, SC_CHUNK)], di_v)
            pltpu.async_copy(table_hbm.at[si_v], rows_v, sem_g).wait()
            pltpu.async_copy(rows_v, out_hbm.at[di_v], sem_s).wait()

    return permute_kernel(table, src_idx, dst_idx)


NFF = 1


def _ffn_body(te_ref, act_ref, x_ref, g_ref, u_ref, d_ref, w_ref, o_ref):
    i = pl.program_id(0)
    f = pl.program_id(1)

    @pl.when(act_ref[i] == 1)
    def _():
        x = x_ref[...]
        g = lax.dot_general(x, g_ref[0], (((1,), (1,)), ((), ())),
                            preferred_element_type=jnp.float32)
        u = lax.dot_general(x, u_ref[0], (((1,), (1,)), ((), ())),
                            preferred_element_type=jnp.float32)
        inter = g * jax.nn.sigmoid(g) * u
        y = lax.dot_general(inter, d_ref[0], (((1,), (1,)), ((), ())),
                            preferred_element_type=jnp.float32)
        y = y * w_ref[0, 0][:, None]

        @pl.when(f == 0)
        def _():
            o_ref[...] = y

        @pl.when(f != 0)
        def _():
            o_ref[...] += y


def _grouped_ffn(x_pad, gate, up, down, w_pad, te, active, nt_max, interpret=False):
    E, FF, H = gate.shape
    R = nt_max * TM
    FB = FF // NFF
    w3 = w_pad.reshape(nt_max, 1, TM)
    grid_spec = pltpu.PrefetchScalarGridSpec(
        num_scalar_prefetch=2,
        grid=(nt_max, NFF),
        in_specs=[
            pl.BlockSpec((TM, H), lambda i, f, te, act: (i, 0)),
            pl.BlockSpec((1, FB, H), lambda i, f, te, act: (te[i], f, 0)),
            pl.BlockSpec((1, FB, H), lambda i, f, te, act: (te[i], f, 0)),
            pl.BlockSpec((1, H, FB), lambda i, f, te, act: (te[i], 0, f)),
            pl.BlockSpec((1, 1, TM), lambda i, f, te, act: (i, 0, 0)),
        ],
        out_specs=pl.BlockSpec((TM, H), lambda i, f, te, act: (i, 0)),
    )
    return pl.pallas_call(
        _ffn_body,
        grid_spec=grid_spec,
        out_shape=jax.ShapeDtypeStruct((R, H), jnp.float32),
        interpret=interpret,
    )(te, active, x_pad, gate, up, down, w3)


def kernel(x, expert_indices, expert_weights, gate_proj, up_proj, down_proj):
    B, S, H = x.shape
    E, FF, _ = gate_proj.shape
    nt_max = S // TM + E

    x_flat = x.reshape(S, H)
    ei = expert_indices.reshape(S).astype(jnp.int32)
    ew = expert_weights.reshape(S)

    te, active, perm, row_p, w_pad = _routing_metadata(ei, ew, E, S, nt_max)
    R = nt_max * TM
    x_padded = _sc_permute_rows(x_flat, perm, row_p, R)
    y_padded = _grouped_ffn(x_padded, gate_proj, up_proj, down_proj,
                            w_pad, te, active, nt_max)
    out = _sc_permute_rows(y_padded, row_p, perm, S)
    return out.reshape(B, S, H)


# speedup vs baseline: 1.0663x; 1.0370x over previous
import functools

import jax
import jax.numpy as jnp
from jax import lax
from jax.experimental import pallas as pl
from jax.experimental.pallas import tpu as pltpu
from jax.experimental.pallas import tpu_sc as plsc

TM = 64
SC_CHUNK = 64
NUM_WORKERS = 32


def _routing_metadata(ei, ew, E, S, nt_max):
    p = jnp.arange(S, dtype=jnp.int32)
    ei_s, perm, ew_s = lax.sort((ei, p, ew), num_keys=1)

    counts = jnp.bincount(ei, length=E).astype(jnp.int32)
    cumc = jnp.cumsum(counts).astype(jnp.int32)
    offs = cumc - counts
    tiles_e = (counts + TM - 1) // TM
    cumtiles = jnp.cumsum(tiles_e).astype(jnp.int32)
    tstart = cumtiles - tiles_e
    num_active = cumtiles[-1]

    tile_ids = jnp.arange(nt_max, dtype=jnp.int32)
    eids = jnp.arange(E, dtype=jnp.int32)
    te = jnp.zeros((nt_max,), jnp.int32).at[tstart].max(eids)
    te = lax.cummax(te, axis=0)
    e_last = jnp.argmax(jnp.where(counts > 0, eids, -1)).astype(jnp.int32)
    te = jnp.where(tile_ids < num_active, te, e_last)
    active = (tile_ids < num_active).astype(jnp.int32)

    base = tstart * TM - offs
    pad_tok = jnp.zeros((S,), jnp.int32).at[offs].max(base)
    row_p = lax.cummax(pad_tok, axis=0) + p

    R = nt_max * TM
    w_pad = jnp.zeros((R,), ew.dtype).at[row_p].set(ew_s)
    return te, active, perm, row_p, w_pad


def _sc_permute_rows(table, src_idx, dst_idx, out_rows):
    S = src_idx.shape[0]
    H = table.shape[1]
    rows_pw = S // NUM_WORKERS
    nch = rows_pw // SC_CHUNK
    mesh = plsc.VectorSubcoreMesh(core_axis_name="c", subcore_axis_name="s")

    @functools.partial(
        pl.kernel,
        out_type=jax.ShapeDtypeStruct((out_rows, H), jnp.float32),
        mesh=mesh,
        scratch_types=[
            pltpu.VMEM((SC_CHUNK,), jnp.int32),
            pltpu.VMEM((SC_CHUNK,), jnp.int32),
            pltpu.VMEM((SC_CHUNK, H), jnp.float32),
            pltpu.SemaphoreType.DMA,
            pltpu.SemaphoreType.DMA,
        ],
    )
    def permute_kernel(table_hbm, si_hbm, di_hbm, out_hbm,
                       si_v, di_v, rows_v, sem_g, sem_s):
        wid = lax.axis_index("s") * 2 + lax.axis_index("c")
        for c in range(nch):
            off = (wid * nch + c) * SC_CHUNK
            pltpu.sync_copy(si_hbm.at[pl.ds(off, SC_CHUNK)], si_v)
            pltpu.sync_copy(di_hbm.at[pl.ds(off, SC_CHUNK)], di_v)
            pltpu.async_copy(table_hbm.at[si_v], rows_v, sem_g).wait()
            pltpu.async_copy(rows_v, out_hbm.at[di_v], sem_s).wait()

    return permute_kernel(table, src_idx, dst_idx)


NFF = 1


def _ffn_body(te_ref, act_ref, x_ref, g_ref, u_ref, d_ref, w_ref, o_ref):
    i = pl.program_id(0)
    f = pl.program_id(1)

    @pl.when(act_ref[i] == 1)
    def _():
        x = x_ref[...]
        g = lax.dot_general(x, g_ref[0], (((1,), (1,)), ((), ())),
                            preferred_element_type=jnp.float32)
        u = lax.dot_general(x, u_ref[0], (((1,), (1,)), ((), ())),
                            preferred_element_type=jnp.float32)
        inter = g * jax.nn.sigmoid(g) * u
        y = lax.dot_general(inter, d_ref[0], (((1,), (1,)), ((), ())),
                            preferred_element_type=jnp.float32)
        y = y * w_ref[0, 0][:, None]

        @pl.when(f == 0)
        def _():
            o_ref[...] = y

        @pl.when(f != 0)
        def _():
            o_ref[...] += y


def _grouped_ffn(x_pad, gate, up, down, w_pad, te, active, nt_max, interpret=False):
    E, FF, H = gate.shape
    R = nt_max * TM
    FB = FF // NFF
    w3 = w_pad.reshape(nt_max, 1, TM)
    grid_spec = pltpu.PrefetchScalarGridSpec(
        num_scalar_prefetch=2,
        grid=(nt_max, NFF),
        in_specs=[
            pl.BlockSpec((TM, H), lambda i, f, te, act: (i, 0)),
            pl.BlockSpec((1, FB, H), lambda i, f, te, act: (te[i], f, 0)),
            pl.BlockSpec((1, FB, H), lambda i, f, te, act: (te[i], f, 0)),
            pl.BlockSpec((1, H, FB), lambda i, f, te, act: (te[i], 0, f)),
            pl.BlockSpec((1, 1, TM), lambda i, f, te, act: (i, 0, 0)),
        ],
        out_specs=pl.BlockSpec((TM, H), lambda i, f, te, act: (i, 0)),
    )
    return pl.pallas_call(
        _ffn_body,
        grid_spec=grid_spec,
        out_shape=jax.ShapeDtypeStruct((R, H), jnp.float32),
        interpret=interpret,
    )(te, active, x_pad, gate, up, down, w3)


def kernel(x, expert_indices, expert_weights, gate_proj, up_proj, down_proj):
    B, S, H = x.shape
    E, FF, _ = gate_proj.shape
    nt_max = S // TM + E

    x_flat = x.reshape(S, H)
    ei = expert_indices.reshape(S).astype(jnp.int32)
    ew = expert_weights.reshape(S)

    te, active, perm, row_p, w_pad = _routing_metadata(ei, ew, E, S, nt_max)
    R = nt_max * TM
    x_padded = _sc_permute_rows(x_flat, perm, row_p, R)
    y_padded = _grouped_ffn(x_padded, gate_proj, up_proj, down_proj,
                            w_pad, te, active, nt_max)
    out = _sc_permute_rows(y_padded, row_p, perm, S)
    return out.reshape(B, S, H)
